# baseline (speedup 1.0000x reference)
import math

import jax
import jax.numpy as jnp
from jax.experimental import pallas as pl
from jax.experimental.pallas import tpu as pltpu

_INV_SQRT2 = 1.0 / math.sqrt(2.0)


def _round_up(a, b):
    return (a + b - 1) // b * b


def _body(x_ref, w1_ref, b1_ref, w2_ref, b2_ref, o_ref):
    xb = x_ref[...].astype(jnp.bfloat16)
    h = jnp.dot(xb, w1_ref[...], preferred_element_type=jnp.float32)
    h = h + b1_ref[...].astype(jnp.float32)
    g = 0.5 * h * (1.0 + jax.lax.erf(h * _INV_SQRT2))
    o = jnp.dot(g.astype(jnp.bfloat16), w2_ref[...],
                preferred_element_type=jnp.float32)
    o_ref[...] = (o + b2_ref[...].astype(jnp.float32)).astype(o_ref.dtype)


def kernel(x, w1, b1, w2, b2):
    B, S, dim = x.shape
    hidden = w1.shape[1]
    M = B * S

    dim_p = _round_up(dim, 128)
    tm = 512 if M >= 512 else _round_up(M, 8)
    M_p = _round_up(M, tm)
    hidden_p = _round_up(hidden, 128)

    x2d = jnp.pad(x.reshape(M, dim), ((0, M_p - M), (0, dim_p - dim)))
    w1p = jnp.pad(w1, ((0, dim_p - dim), (0, hidden_p - hidden))).astype(jnp.bfloat16)
    b1p = jnp.pad(b1, (0, hidden_p - hidden)).reshape(1, hidden_p)
    w2p = jnp.pad(w2, ((0, hidden_p - hidden), (0, dim_p - dim))).astype(jnp.bfloat16)
    b2p = jnp.pad(b2, (0, dim_p - dim)).reshape(1, dim_p)

    out2d = pl.pallas_call(
        _body,
        out_shape=jax.ShapeDtypeStruct((M_p, dim_p), x.dtype),
        grid=(M_p // tm,),
        in_specs=[
            pl.BlockSpec((tm, dim_p), lambda i: (i, 0)),
            pl.BlockSpec((dim_p, hidden_p), lambda i: (0, 0)),
            pl.BlockSpec((1, hidden_p), lambda i: (0, 0)),
            pl.BlockSpec((hidden_p, dim_p), lambda i: (0, 0)),
            pl.BlockSpec((1, dim_p), lambda i: (0, 0)),
        ],
        out_specs=pl.BlockSpec((tm, dim_p), lambda i: (i, 0)),
        compiler_params=pltpu.CompilerParams(
            dimension_semantics=("arbitrary",),
            vmem_limit_bytes=100 * 1024 * 1024,
        ),
    )(x2d, w1p, b1p, w2p, b2p)

    return out2d[:M, :dim].reshape(B, S, dim)


# speedup vs baseline: 1.1039x; 1.1039x over previous
import math

import jax
import jax.numpy as jnp
from jax.experimental import pallas as pl
from jax.experimental.pallas import tpu as pltpu

_INV_SQRT2 = 1.0 / math.sqrt(2.0)


def _round_up(a, b):
    return (a + b - 1) // b * b


def _make_body(n_tiles, tm, dim_p):
    def _body(x_hbm, w1_ref, b1_ref, w2_ref, b2_ref, o_hbm, w1b, w2b):
        w1b[...] = w1_ref[...].astype(jnp.bfloat16)
        w2b[...] = w2_ref[...].astype(jnp.bfloat16)

        def _tile(x_ref, o_ref):
            xb = x_ref[...].astype(jnp.bfloat16)
            h = jnp.dot(xb, w1b[...], preferred_element_type=jnp.float32)
            h = h + b1_ref[...].astype(jnp.float32)
            g = 0.5 * h * (1.0 + jax.lax.erf(h * _INV_SQRT2))
            o = jnp.dot(g.astype(jnp.bfloat16), w2b[...],
                        preferred_element_type=jnp.float32)
            o_ref[...] = (o + b2_ref[...].astype(jnp.float32)).astype(o_ref.dtype)

        pltpu.emit_pipeline(
            _tile,
            grid=(n_tiles,),
            in_specs=[pl.BlockSpec((tm, dim_p), lambda i: (i, 0))],
            out_specs=[pl.BlockSpec((tm, dim_p), lambda i: (i, 0))],
        )(x_hbm, o_hbm)

    return _body


def kernel(x, w1, b1, w2, b2):
    B, S, dim = x.shape
    hidden = w1.shape[1]
    M = B * S

    dim_p = _round_up(dim, 128)
    tm = 512 if M >= 512 else _round_up(M, 8)
    M_p = _round_up(M, tm)
    hidden_p = _round_up(hidden, 128)
    n_tiles = M_p // tm

    x2d = jnp.pad(x.reshape(M, dim), ((0, M_p - M), (0, dim_p - dim)))
    w1p = jnp.pad(w1, ((0, dim_p - dim), (0, hidden_p - hidden)))
    b1p = jnp.pad(b1, (0, hidden_p - hidden)).reshape(1, hidden_p)
    w2p = jnp.pad(w2, ((0, hidden_p - hidden), (0, dim_p - dim)))
    b2p = jnp.pad(b2, (0, dim_p - dim)).reshape(1, dim_p)

    out2d = pl.pallas_call(
        _make_body(n_tiles, tm, dim_p),
        out_shape=jax.ShapeDtypeStruct((M_p, dim_p), x.dtype),
        in_specs=[
            pl.BlockSpec(memory_space=pl.ANY),
            pl.BlockSpec((dim_p, hidden_p), lambda: (0, 0)),
            pl.BlockSpec((1, hidden_p), lambda: (0, 0)),
            pl.BlockSpec((hidden_p, dim_p), lambda: (0, 0)),
            pl.BlockSpec((1, dim_p), lambda: (0, 0)),
        ],
        out_specs=pl.BlockSpec(memory_space=pl.ANY),
        scratch_shapes=[
            pltpu.VMEM((dim_p, hidden_p), jnp.bfloat16),
            pltpu.VMEM((hidden_p, dim_p), jnp.bfloat16),
        ],
        compiler_params=pltpu.CompilerParams(
            vmem_limit_bytes=100 * 1024 * 1024,
        ),
    )(x2d, w1p, b1p, w2p, b2p)

    return out2d[:M, :dim].reshape(B, S, dim)


# speedup vs baseline: 1.1151x; 1.0101x over previous
import math

import jax
import jax.numpy as jnp
from jax.experimental import pallas as pl
from jax.experimental.pallas import tpu as pltpu

_INV_SQRT2 = 1.0 / math.sqrt(2.0)


def _round_up(a, b):
    return (a + b - 1) // b * b


def _make_body(n_tiles, tm, dim_p):
    def _body(x_hbm, w1_ref, b1_ref, w2_ref, b2_ref, o_hbm, w1b, w2b):
        w1b[...] = w1_ref[...].astype(jnp.bfloat16)
        w2b[...] = w2_ref[...].astype(jnp.bfloat16)

        def _tile(x_ref, o_ref):
            xb = x_ref[...].astype(jnp.bfloat16)
            h = jnp.dot(xb, w1b[...], preferred_element_type=jnp.float32)
            h = h + b1_ref[...].astype(jnp.float32)
            g = 0.5 * h * (1.0 + jax.lax.erf(h * _INV_SQRT2))
            o = jnp.dot(g.astype(jnp.bfloat16), w2b[...],
                        preferred_element_type=jnp.float32)
            o_ref[...] = (o + b2_ref[...].astype(jnp.float32)).astype(o_ref.dtype)

        pltpu.emit_pipeline(
            _tile,
            grid=(n_tiles,),
            in_specs=[pl.BlockSpec((tm, dim_p), lambda i: (i, 0))],
            out_specs=[pl.BlockSpec((tm, dim_p), lambda i: (i, 0))],
        )(x_hbm, o_hbm)

    return _body


def kernel(x, w1, b1, w2, b2):
    B, S, dim = x.shape
    hidden = w1.shape[1]
    M = B * S

    dim_p = _round_up(dim, 128)
    tm = 1024 if M >= 1024 else _round_up(M, 8)
    M_p = _round_up(M, tm)
    hidden_p = _round_up(hidden, 128)
    n_tiles = M_p // tm

    x2d = jnp.pad(x.reshape(M, dim), ((0, M_p - M), (0, dim_p - dim)))
    w1p = jnp.pad(w1, ((0, dim_p - dim), (0, hidden_p - hidden)))
    b1p = jnp.pad(b1, (0, hidden_p - hidden)).reshape(1, hidden_p)
    w2p = jnp.pad(w2, ((0, hidden_p - hidden), (0, dim_p - dim)))
    b2p = jnp.pad(b2, (0, dim_p - dim)).reshape(1, dim_p)

    out2d = pl.pallas_call(
        _make_body(n_tiles, tm, dim_p),
        out_shape=jax.ShapeDtypeStruct((M_p, dim_p), x.dtype),
        in_specs=[
            pl.BlockSpec(memory_space=pl.ANY),
            pl.BlockSpec((dim_p, hidden_p), lambda: (0, 0)),
            pl.BlockSpec((1, hidden_p), lambda: (0, 0)),
            pl.BlockSpec((hidden_p, dim_p), lambda: (0, 0)),
            pl.BlockSpec((1, dim_p), lambda: (0, 0)),
        ],
        out_specs=pl.BlockSpec(memory_space=pl.ANY),
        scratch_shapes=[
            pltpu.VMEM((dim_p, hidden_p), jnp.bfloat16),
            pltpu.VMEM((hidden_p, dim_p), jnp.bfloat16),
        ],
        compiler_params=pltpu.CompilerParams(
            vmem_limit_bytes=100 * 1024 * 1024,
        ),
    )(x2d, w1p, b1p, w2p, b2p)

    return out2d[:M, :dim].reshape(B, S, dim)
